# baseline (speedup 1.0000x reference)
import jax
import jax.numpy as jnp
from jax import lax
from jax.experimental import pallas as pl
from jax.experimental.pallas import tpu as pltpu
from jax.experimental.pallas import tpu_sc as plsc

N = 10000
D = 256
OUT = 64
L = 10
EPS = 1e-5

NC = 2
NS = 16
NT = NC * NS
BUCKET = 320
ACC_ROWS = 336
DUMP = 328
CH = 16
GS = float(1.0 / (1.0 + EPS) ** 0.5)


def _mesh():
    return plsc.VectorSubcoreMesh(core_axis_name="c", subcore_axis_name="s",
                                  num_cores=NC, num_subcores=NS)


_SC_PARAMS = pltpu.CompilerParams(needs_layout_passes=False)


def _partition_body(src_hbm, dst_hbm, srcs_out, dstl_out, offs_out, cnts_out,
                    deg_out, src_v, dst_v, so, do_, hist, cnt_v, offs_v):
    E = src_hbm.shape[0]
    ept = E // NT
    slot = ept + NT * CH
    c = lax.axis_index("c")
    s = lax.axis_index("s")
    k = c * NS + s

    pltpu.sync_copy(src_hbm.at[pl.ds(k * ept, ept)], src_v.at[pl.ds(0, ept)])
    pltpu.sync_copy(dst_hbm.at[pl.ds(k * ept, ept)], dst_v.at[pl.ds(0, ept)])

    zeros_i = jnp.zeros((16,), jnp.int32)
    dump_i = jnp.full((16,), DUMP, jnp.int32)
    zeros_f = jnp.zeros((16,), jnp.float32)
    ones_f = jnp.ones((16,), jnp.float32)
    ones_i = jnp.ones((16,), jnp.int32)
    iota = lax.iota(jnp.int32, 16)

    def init_hist(i, _):
        hist[pl.ds(i * 16, 16)] = zeros_f
        return 0
    lax.fori_loop(0, N // 16, init_hist, 0)

    def init_slots(i, _):
        so[pl.ds(i * 16, 16)] = (i * 16 + iota) & 8191
        do_[pl.ds(i * 16, 16)] = dump_i
        return 0
    lax.fori_loop(0, slot // 16, init_slots, 0)

    cnt_v[pl.ds(0, 16)] = zeros_i
    cnt_v[pl.ds(16, 16)] = zeros_i

    def step(i, _):
        dstv = dst_v[pl.ds(i * 16, 16)]
        valid = iota < (ept - i * 16)
        dstv = jnp.where(valid, dstv, 0)
        plsc.addupdate_scatter(hist, [dstv], ones_f, mask=valid)
        plsc.addupdate_scatter(cnt_v, [dstv // BUCKET], ones_i, mask=valid)
        return 0
    lax.fori_loop(0, (ept + 15) // 16, step, 0)

    c0 = cnt_v[pl.ds(0, 16)]
    c1 = cnt_v[pl.ds(16, 16)]
    p0 = (c0 + (CH - 1)) & (-CH)
    p1 = (c1 + (CH - 1)) & (-CH)
    i0 = plsc.cumsum(p0)
    i1 = plsc.cumsum(p1) + jnp.max(i0)
    offs_v[pl.ds(0, 16)] = i0 - p0
    offs_v[pl.ds(16, 16)] = i1 - p1
    cnt_v[pl.ds(0, 16)] = p0
    cnt_v[pl.ds(16, 16)] = p1

    nsteps = (ept + 15) // 16

    def per_bucket(b, _):
        o0 = offs_v[pl.ds(b, 16)][0]

        def sweep(i, o):
            srcv = src_v[pl.ds(i * 16, 16)]
            dstv = dst_v[pl.ds(i * 16, 16)]
            valid = iota < (ept - i * 16)
            dstv = jnp.where(valid, dstv, 0)
            m = valid & ((dstv // BUCKET) == b)
            plsc.store_compressed(so.at[pl.ds(o, 16)], srcv, mask=m)
            plsc.store_compressed(do_.at[pl.ds(o, 16)], dstv - b * BUCKET,
                                  mask=m)
            return o + jnp.max(plsc.all_reduce_population_count(m))
        lax.fori_loop(0, nsteps, sweep, o0)
        return 0
    lax.fori_loop(0, 32, per_bucket, 0)

    pltpu.sync_copy(so, srcs_out.at[pl.ds(k * slot, slot)])
    pltpu.sync_copy(do_, dstl_out.at[pl.ds(k * slot, slot)])
    pltpu.sync_copy(offs_v.at[pl.ds(0, 32)], offs_out.at[pl.ds(k * 32, 32)])
    pltpu.sync_copy(cnt_v, cnts_out.at[pl.ds(k * 32, 32)])
    pltpu.sync_copy(hist, deg_out.at[k])


def _make_partition(E):
    ept = E // NT
    slot = ept + NT * CH
    return pl.kernel(
        _partition_body,
        out_type=[
            jax.ShapeDtypeStruct((NT * slot,), jnp.int32),
            jax.ShapeDtypeStruct((NT * slot,), jnp.int32),
            jax.ShapeDtypeStruct((NT * 32,), jnp.int32),
            jax.ShapeDtypeStruct((NT * 32,), jnp.int32),
            jax.ShapeDtypeStruct((NT, N), jnp.float32),
        ],
        mesh=_mesh(),
        compiler_params=_SC_PARAMS,
        scratch_types=[
            pltpu.VMEM((ept + 16,), jnp.int32),
            pltpu.VMEM((ept + 16,), jnp.int32),
            pltpu.VMEM((slot,), jnp.int32),
            pltpu.VMEM((slot,), jnp.int32),
            pltpu.VMEM((N,), jnp.float32),
            pltpu.VMEM((32,), jnp.int32),
            pltpu.VMEM((48,), jnp.int32),
        ],
    )


MAXCH = 10240


def _make_chunks(eslot):
    def body(offs_hbm, cnts_hbm, coffs_out, ncts_out, tabs, coffs_v):
        c = lax.axis_index("c")
        s = lax.axis_index("s")
        w = c * NS + s

        pltpu.sync_copy(offs_hbm, tabs.at[pl.ds(0, NT * 32)])
        pltpu.sync_copy(cnts_hbm, tabs.at[pl.ds(NT * 32, NT * 32)])

        iota = lax.iota(jnp.int32, 16)
        idx0 = iota * 32 + w
        colo0 = plsc.load_gather(tabs, [idx0])
        colo1 = plsc.load_gather(tabs, [idx0 + 512])
        coln0 = plsc.load_gather(tabs, [idx0 + NT * 32])
        coln1 = plsc.load_gather(tabs, [idx0 + NT * 32 + 512])

        def _extr(v, i):
            return jnp.max(jnp.where(iota == i, v, 0))

        def per_tile(t, ptr):
            o = jnp.where(t < NS, _extr(colo0, t), _extr(colo1, t - NS))
            n = jnp.where(t < NS, _extr(coln0, t), _extr(coln1, t - NS))
            base = t * eslot + o
            nch = n // CH

            def grp(g, pp):
                ii = g * 16 + iota
                plsc.store_scatter(coffs_v, [pp + ii], base + ii * CH,
                                   mask=ii < nch)
                return pp
            lax.fori_loop(0, (nch + 15) // 16, grp, ptr)
            return ptr + nch
        nct = lax.fori_loop(0, NT, per_tile, jnp.int32(0))

        pltpu.sync_copy(coffs_v, coffs_out.at[pl.ds(w * MAXCH, MAXCH)])
        tabs[pl.ds(0, 16)] = jnp.zeros((16,), jnp.int32) + nct
        pltpu.sync_copy(tabs.at[pl.ds(0, 16)], ncts_out.at[pl.ds(w * 16, 16)])

    return pl.kernel(
        body,
        out_type=[
            jax.ShapeDtypeStruct((NT * MAXCH,), jnp.int32),
            jax.ShapeDtypeStruct((NT * 16,), jnp.int32),
        ],
        mesh=_mesh(),
        compiler_params=_SC_PARAMS,
        scratch_types=[
            pltpu.VMEM((2 * NT * 32 + 16,), jnp.int32),
            pltpu.VMEM((MAXCH,), jnp.int32),
        ],
    )


def _agg_body(y_hbm, srcs_hbm, dstl_hbm, coffs_hbm, ncts_hbm, agg_hbm,
              acc, rows0, rows1, rows2, sidx0, sidx1, sidx2, coffs_v, cnt_v,
              isem0, isem1, isem2, gsem0, gsem1, gsem2):
    c = lax.axis_index("c")
    s = lax.axis_index("s")
    w = c * NS + s
    base_row = w * BUCKET

    pltpu.sync_copy(coffs_hbm.at[pl.ds(w * MAXCH, MAXCH)], coffs_v)
    pltpu.sync_copy(ncts_hbm.at[pl.ds(w * 16, 16)], cnt_v)
    nct = jnp.max(cnt_v[...])

    last = N - BUCKET * (NT - 1)
    extra = BUCKET - last
    pltpu.sync_copy(y_hbm.at[pl.ds(base_row, last)], acc.at[pl.ds(0, last)])
    @pl.when(w < NT - 1)
    def _():
        pltpu.sync_copy(y_hbm.at[pl.ds(base_row + last, extra)],
                        acc.at[pl.ds(last, extra)])

    rows_ = (rows0, rows1, rows2)
    ibuf_ = (sidx0, sidx1, sidx2)
    isem_ = (isem0, isem1, isem2)
    gsem_ = (gsem0, gsem1, gsem2)

    def _start_idx(j, b):
        v = plsc.load_gather(coffs_v, [jnp.full((16,), j, jnp.int32)])
        off = pl.multiple_of(v[0], 8)
        pltpu.async_copy(srcs_hbm.at[pl.ds(off, CH)],
                         ibuf_[b].at[pl.ds(0, CH)], isem_[b])
        pltpu.async_copy(dstl_hbm.at[pl.ds(off, CH)],
                         ibuf_[b].at[pl.ds(CH, CH)], isem_[b])

    def _wait_idx(b):
        pltpu.make_async_copy(srcs_hbm.at[pl.ds(0, 2 * CH)], ibuf_[b],
                              isem_[b]).wait()

    @pl.when(nct > 0)
    def _():
        _start_idx(0, 0)

    def triple(i, _):
        for b in (0, 1, 2):
            j = 3 * i + b
            ab = (b + 1) % 3

            @pl.when(j < nct)
            def _():
                _wait_idx(b)
                pltpu.async_copy(y_hbm.at[ibuf_[b].at[pl.ds(0, CH)]],
                                 rows_[b], gsem_[b])

            if b == 0:
                @pl.when(j == 0)
                def _():
                    @pl.when(nct > 1)
                    def _():
                        _start_idx(1, 1)
                    @pl.when(nct > 2)
                    def _():
                        _start_idx(2, 2)

            @pl.when((j >= 2) & (j <= nct + 1))
            def _():
                pltpu.make_async_copy(y_hbm.at[pl.ds(0, CH)], rows_[ab],
                                      gsem_[ab]).wait()
                dvecs = [ibuf_[ab][pl.ds(CH + k * 16, 16)]
                         for k in range(CH // 16)]

                @pl.when(j + 1 < nct)
                def _():
                    _start_idx(j + 1, ab)

                rnb = rows_[ab]
                for k in range(CH // 16):
                    dvec = dvecs[k]
                    for e in range(16):
                        dl = dvec[e]
                        r = k * 16 + e
                        for jj in range(D // 16):
                            plsc.addupdate(acc.at[dl, pl.ds(jj * 16, 16)],
                                           rnb[r, pl.ds(jj * 16, 16)])
        return 0
    lax.fori_loop(0, (nct + 4) // 3, triple, 0)

    pltpu.sync_copy(acc.at[pl.ds(0, last)],
                    agg_hbm.at[pl.ds(base_row, last)])
    @pl.when(w < NT - 1)
    def _():
        pltpu.sync_copy(acc.at[pl.ds(last, extra)],
                        agg_hbm.at[pl.ds(base_row + last, extra)])


def _make_agg():
    return pl.kernel(
        _agg_body,
        out_type=jax.ShapeDtypeStruct((N, D), jnp.float32),
        mesh=_mesh(),
        compiler_params=_SC_PARAMS,
        scratch_types=[
            pltpu.VMEM((ACC_ROWS, D), jnp.float32),
            pltpu.VMEM((CH, D), jnp.float32),
            pltpu.VMEM((CH, D), jnp.float32),
            pltpu.VMEM((CH, D), jnp.float32),
            pltpu.VMEM((2 * CH,), jnp.int32),
            pltpu.VMEM((2 * CH,), jnp.int32),
            pltpu.VMEM((2 * CH,), jnp.int32),
            pltpu.VMEM((MAXCH,), jnp.int32),
            pltpu.VMEM((16,), jnp.int32),
            pltpu.SemaphoreType.DMA,
            pltpu.SemaphoreType.DMA,
            pltpu.SemaphoreType.DMA,
            pltpu.SemaphoreType.DMA,
            pltpu.SemaphoreType.DMA,
            pltpu.SemaphoreType.DMA,
        ],
    )


def _dinv_body(dp_ref, o_ref):
    o_ref[...] = lax.rsqrt(jnp.sum(dp_ref[...], axis=0, keepdims=True) + 1.0)


def _mm0_body(x_ref, w_ref, dinv_ref, y_ref):
    y_ref[...] = jnp.dot(x_ref[...], w_ref[...],
                         preferred_element_type=jnp.float32) * dinv_ref[...]


def _fused_body(acc_ref, dinv_ref, b_ref, g_ref, bt_ref, w_ref, y_ref):
    dinv = dinv_ref[...]
    t = acc_ref[...] * dinv + b_ref[...]
    t = jnp.maximum(t, 0.0) * (g_ref[...] * GS) + bt_ref[...]
    y_ref[...] = jnp.dot(t, w_ref[...],
                         preferred_element_type=jnp.float32) * dinv


def _final_body(acc_ref, dinv_ref, b_ref, g_ref, bt_ref, w_ref, lb_ref, o_ref):
    t = acc_ref[...] * dinv_ref[...] + b_ref[...]
    t = jnp.maximum(t, 0.0) * (g_ref[...] * GS) + bt_ref[...]
    o_ref[...] = jnp.dot(t, w_ref[...],
                         preferred_element_type=jnp.float32) + lb_ref[...]


BM = 400
GRID = (N // BM,)


def _row_spec(width):
    return pl.BlockSpec((BM, width), lambda i: (i, 0))


def _rep_spec(shape):
    return pl.BlockSpec(shape, lambda i: (0,) * len(shape))


def kernel(x, edge_index, conv_W, conv_b, bn_gamma, bn_beta, lin_W, lin_b):
    src = edge_index[0]
    dst = edge_index[1]
    E = src.shape[0]

    srcs_p, dstl_p, offs_p, cnts_p, deg_p = _make_partition(E)(src, dst)
    coffs_p, ncts_p = _make_chunks(E // NT + NT * CH)(offs_p, cnts_p)

    dinv_row = pl.pallas_call(
        _dinv_body,
        out_shape=jax.ShapeDtypeStruct((1, N), jnp.float32),
    )(deg_p)
    dinv = dinv_row.reshape(N, 1)

    mm0 = pl.pallas_call(
        _mm0_body,
        grid=GRID,
        in_specs=[_row_spec(D), _rep_spec((D, D)), _row_spec(1)],
        out_specs=_row_spec(D),
        out_shape=jax.ShapeDtypeStruct((N, D), jnp.float32),
    )
    fused = pl.pallas_call(
        _fused_body,
        grid=GRID,
        in_specs=[_row_spec(D), _row_spec(1), _rep_spec((1, D)),
                  _rep_spec((1, D)), _rep_spec((1, D)), _rep_spec((D, D))],
        out_specs=_row_spec(D),
        out_shape=jax.ShapeDtypeStruct((N, D), jnp.float32),
    )
    final = pl.pallas_call(
        _final_body,
        grid=GRID,
        in_specs=[_row_spec(D), _row_spec(1), _rep_spec((1, D)),
                  _rep_spec((1, D)), _rep_spec((1, D)), _rep_spec((D, OUT)),
                  _rep_spec((1, OUT))],
        out_specs=_row_spec(OUT),
        out_shape=jax.ShapeDtypeStruct((N, OUT), jnp.float32),
    )
    agg_call = _make_agg()

    y = mm0(x, conv_W[0], dinv)
    for i in range(L):
        acc = agg_call(y, srcs_p, dstl_p, coffs_p, ncts_p)
        if i < L - 1:
            y = fused(acc, dinv, conv_b[i].reshape(1, D),
                      bn_gamma[i].reshape(1, D), bn_beta[i].reshape(1, D),
                      conv_W[i + 1])
        else:
            out = final(acc, dinv, conv_b[i].reshape(1, D),
                        bn_gamma[i].reshape(1, D), bn_beta[i].reshape(1, D),
                        lin_W, lin_b.reshape(1, OUT))
    return out


# speedup vs baseline: 1.0531x; 1.0531x over previous
import jax
import jax.numpy as jnp
from jax import lax
from jax.experimental import pallas as pl
from jax.experimental.pallas import tpu as pltpu
from jax.experimental.pallas import tpu_sc as plsc

N = 10000
D = 256
OUT = 64
L = 10
EPS = 1e-5

NC = 2
NS = 16
NT = NC * NS
BUCKET = 320
ACC_ROWS = 336
DUMP = 328
CH = 16
GS = float(1.0 / (1.0 + EPS) ** 0.5)


def _mesh():
    return plsc.VectorSubcoreMesh(core_axis_name="c", subcore_axis_name="s",
                                  num_cores=NC, num_subcores=NS)


_SC_PARAMS = pltpu.CompilerParams(needs_layout_passes=False)


def _partition_body(src_hbm, dst_hbm, srcs_out, dstl_out, offs_out, cnts_out,
                    deg_out, src_v, dst_v, so, do_, hist, cnt_v, offs_v):
    E = src_hbm.shape[0]
    ept = E // NT
    slot = ept + NT * CH
    c = lax.axis_index("c")
    s = lax.axis_index("s")
    k = c * NS + s

    pltpu.sync_copy(src_hbm.at[pl.ds(k * ept, ept)], src_v.at[pl.ds(0, ept)])
    pltpu.sync_copy(dst_hbm.at[pl.ds(k * ept, ept)], dst_v.at[pl.ds(0, ept)])

    zeros_i = jnp.zeros((16,), jnp.int32)
    dump_i = jnp.full((16,), DUMP, jnp.int32)
    zeros_f = jnp.zeros((16,), jnp.float32)
    ones_f = jnp.ones((16,), jnp.float32)
    ones_i = jnp.ones((16,), jnp.int32)
    iota = lax.iota(jnp.int32, 16)

    def init_hist(i, _):
        hist[pl.ds(i * 16, 16)] = zeros_f
        return 0
    lax.fori_loop(0, N // 16, init_hist, 0)

    def init_slots(i, _):
        so[pl.ds(i * 16, 16)] = (i * 16 + iota) & 8191
        do_[pl.ds(i * 16, 16)] = dump_i
        return 0
    lax.fori_loop(0, slot // 16, init_slots, 0)

    cnt_v[pl.ds(0, 16)] = zeros_i
    cnt_v[pl.ds(16, 16)] = zeros_i

    def step(i, _):
        dstv = dst_v[pl.ds(i * 16, 16)]
        valid = iota < (ept - i * 16)
        dstv = jnp.where(valid, dstv, 0)
        plsc.addupdate_scatter(hist, [dstv], ones_f, mask=valid)
        plsc.addupdate_scatter(cnt_v, [dstv // BUCKET], ones_i, mask=valid)
        return 0
    lax.fori_loop(0, (ept + 15) // 16, step, 0)

    c0 = cnt_v[pl.ds(0, 16)]
    c1 = cnt_v[pl.ds(16, 16)]
    p0 = (c0 + (CH - 1)) & (-CH)
    p1 = (c1 + (CH - 1)) & (-CH)
    i0 = plsc.cumsum(p0)
    i1 = plsc.cumsum(p1) + jnp.max(i0)
    offs_v[pl.ds(0, 16)] = i0 - p0
    offs_v[pl.ds(16, 16)] = i1 - p1
    cnt_v[pl.ds(0, 16)] = p0
    cnt_v[pl.ds(16, 16)] = p1

    nsteps = (ept + 15) // 16

    def per_bucket(b, _):
        o0 = offs_v[pl.ds(b, 16)][0]

        def sweep(i, o):
            srcv = src_v[pl.ds(i * 16, 16)]
            dstv = dst_v[pl.ds(i * 16, 16)]
            valid = iota < (ept - i * 16)
            dstv = jnp.where(valid, dstv, 0)
            m = valid & ((dstv // BUCKET) == b)
            plsc.store_compressed(so.at[pl.ds(o, 16)], srcv, mask=m)
            plsc.store_compressed(do_.at[pl.ds(o, 16)], dstv - b * BUCKET,
                                  mask=m)
            return o + jnp.max(plsc.all_reduce_population_count(m))
        lax.fori_loop(0, nsteps, sweep, o0)
        return 0
    lax.fori_loop(0, 32, per_bucket, 0)

    pltpu.sync_copy(so, srcs_out.at[pl.ds(k * slot, slot)])
    pltpu.sync_copy(do_, dstl_out.at[pl.ds(k * slot, slot)])
    pltpu.sync_copy(offs_v.at[pl.ds(0, 32)], offs_out.at[pl.ds(k * 32, 32)])
    pltpu.sync_copy(cnt_v, cnts_out.at[pl.ds(k * 32, 32)])
    pltpu.sync_copy(hist, deg_out.at[k])


def _make_partition(E):
    ept = E // NT
    slot = ept + NT * CH
    return pl.kernel(
        _partition_body,
        out_type=[
            jax.ShapeDtypeStruct((NT * slot,), jnp.int32),
            jax.ShapeDtypeStruct((NT * slot,), jnp.int32),
            jax.ShapeDtypeStruct((NT * 32,), jnp.int32),
            jax.ShapeDtypeStruct((NT * 32,), jnp.int32),
            jax.ShapeDtypeStruct((NT, N), jnp.float32),
        ],
        mesh=_mesh(),
        compiler_params=_SC_PARAMS,
        scratch_types=[
            pltpu.VMEM((ept + 16,), jnp.int32),
            pltpu.VMEM((ept + 16,), jnp.int32),
            pltpu.VMEM((slot,), jnp.int32),
            pltpu.VMEM((slot,), jnp.int32),
            pltpu.VMEM((N,), jnp.float32),
            pltpu.VMEM((32,), jnp.int32),
            pltpu.VMEM((48,), jnp.int32),
        ],
    )


MAXCH = 10240


def _make_chunks(eslot):
    def body(offs_hbm, cnts_hbm, coffs_out, ncts_out, tabs, coffs_v):
        c = lax.axis_index("c")
        s = lax.axis_index("s")
        w = c * NS + s

        pltpu.sync_copy(offs_hbm, tabs.at[pl.ds(0, NT * 32)])
        pltpu.sync_copy(cnts_hbm, tabs.at[pl.ds(NT * 32, NT * 32)])

        iota = lax.iota(jnp.int32, 16)
        idx0 = iota * 32 + w
        colo0 = plsc.load_gather(tabs, [idx0])
        colo1 = plsc.load_gather(tabs, [idx0 + 512])
        coln0 = plsc.load_gather(tabs, [idx0 + NT * 32])
        coln1 = plsc.load_gather(tabs, [idx0 + NT * 32 + 512])

        def _extr(v, i):
            return jnp.max(jnp.where(iota == i, v, 0))

        def per_tile(t, ptr):
            o = jnp.where(t < NS, _extr(colo0, t), _extr(colo1, t - NS))
            n = jnp.where(t < NS, _extr(coln0, t), _extr(coln1, t - NS))
            base = t * eslot + o
            nch = n // CH

            def grp(g, pp):
                ii = g * 16 + iota
                plsc.store_scatter(coffs_v, [pp + ii], base + ii * CH,
                                   mask=ii < nch)
                return pp
            lax.fori_loop(0, (nch + 15) // 16, grp, ptr)
            return ptr + nch
        nct = lax.fori_loop(0, NT, per_tile, jnp.int32(0))

        pltpu.sync_copy(coffs_v, coffs_out.at[pl.ds(w * MAXCH, MAXCH)])
        tabs[pl.ds(0, 16)] = jnp.zeros((16,), jnp.int32) + nct
        pltpu.sync_copy(tabs.at[pl.ds(0, 16)], ncts_out.at[pl.ds(w * 16, 16)])

    return pl.kernel(
        body,
        out_type=[
            jax.ShapeDtypeStruct((NT * MAXCH,), jnp.int32),
            jax.ShapeDtypeStruct((NT * 16,), jnp.int32),
        ],
        mesh=_mesh(),
        compiler_params=_SC_PARAMS,
        scratch_types=[
            pltpu.VMEM((2 * NT * 32 + 16,), jnp.int32),
            pltpu.VMEM((MAXCH,), jnp.int32),
        ],
    )


def _agg_body(y_hbm, srcs_hbm, dstl_hbm, coffs_hbm, ncts_hbm, agg_hbm,
              acc, rows0, rows1, sidx0, sidx1, coffs_v, cnt_v,
              isem0, isem1, gsem0, gsem1):
    c = lax.axis_index("c")
    s = lax.axis_index("s")
    w = c * NS + s
    base_row = w * BUCKET

    pltpu.sync_copy(coffs_hbm.at[pl.ds(w * MAXCH, MAXCH)], coffs_v)
    pltpu.sync_copy(ncts_hbm.at[pl.ds(w * 16, 16)], cnt_v)
    nct = jnp.max(cnt_v[...])

    last = N - BUCKET * (NT - 1)
    extra = BUCKET - last
    pltpu.sync_copy(y_hbm.at[pl.ds(base_row, last)], acc.at[pl.ds(0, last)])
    @pl.when(w < NT - 1)
    def _():
        pltpu.sync_copy(y_hbm.at[pl.ds(base_row + last, extra)],
                        acc.at[pl.ds(last, extra)])

    rows_ = (rows0, rows1)
    ibuf_ = (sidx0, sidx1)
    isem_ = (isem0, isem1)
    gsem_ = (gsem0, gsem1)

    def _start_idx(j, b):
        v = plsc.load_gather(coffs_v, [jnp.full((16,), j, jnp.int32)])
        off = pl.multiple_of(v[0], 8)
        pltpu.async_copy(srcs_hbm.at[pl.ds(off, CH)],
                         ibuf_[b].at[pl.ds(0, CH)], isem_[b])
        pltpu.async_copy(dstl_hbm.at[pl.ds(off, CH)],
                         ibuf_[b].at[pl.ds(CH, CH)], isem_[b])

    def _wait_idx(b):
        pltpu.make_async_copy(srcs_hbm.at[pl.ds(0, 2 * CH)], ibuf_[b],
                              isem_[b]).wait()

    @pl.when(nct > 0)
    def _():
        _start_idx(0, 0)

    def pair(i, _):
        for b in (0, 1):
            j = 2 * i + b
            nb = 1 - b

            @pl.when(j < nct)
            def _():
                _wait_idx(b)
                pltpu.async_copy(y_hbm.at[ibuf_[b].at[pl.ds(0, CH)]],
                                 rows_[b], gsem_[b])

            if b == 0:
                @pl.when((j == 0) & (nct > 1))
                def _():
                    _start_idx(1, 1)

            @pl.when((j > 0) & (j <= nct))
            def _():
                pltpu.make_async_copy(y_hbm.at[pl.ds(0, CH)], rows_[nb],
                                      gsem_[nb]).wait()
                dvecs = [ibuf_[nb][pl.ds(CH + k * 16, 16)]
                         for k in range(CH // 16)]

                @pl.when(j + 1 < nct)
                def _():
                    _start_idx(j + 1, nb)

                rnb = rows_[nb]
                for k in range(CH // 16):
                    dvec = dvecs[k]
                    for e in range(16):
                        dl = dvec[e]
                        r = k * 16 + e
                        for jj in range(D // 16):
                            plsc.addupdate(acc.at[dl, pl.ds(jj * 16, 16)],
                                           rnb[r, pl.ds(jj * 16, 16)])
        return 0
    lax.fori_loop(0, (nct + 2) // 2, pair, 0)

    pltpu.sync_copy(acc.at[pl.ds(0, last)],
                    agg_hbm.at[pl.ds(base_row, last)])
    @pl.when(w < NT - 1)
    def _():
        pltpu.sync_copy(acc.at[pl.ds(last, extra)],
                        agg_hbm.at[pl.ds(base_row + last, extra)])


def _make_agg():
    return pl.kernel(
        _agg_body,
        out_type=jax.ShapeDtypeStruct((N, D), jnp.float32),
        mesh=_mesh(),
        compiler_params=_SC_PARAMS,
        scratch_types=[
            pltpu.VMEM((ACC_ROWS, D), jnp.float32),
            pltpu.VMEM((CH, D), jnp.float32),
            pltpu.VMEM((CH, D), jnp.float32),
            pltpu.VMEM((2 * CH,), jnp.int32),
            pltpu.VMEM((2 * CH,), jnp.int32),
            pltpu.VMEM((MAXCH,), jnp.int32),
            pltpu.VMEM((16,), jnp.int32),
            pltpu.SemaphoreType.DMA,
            pltpu.SemaphoreType.DMA,
            pltpu.SemaphoreType.DMA,
            pltpu.SemaphoreType.DMA,
        ],
    )


def _dinv_body(dp_ref, o_ref):
    o_ref[...] = lax.rsqrt(jnp.sum(dp_ref[...], axis=0, keepdims=True) + 1.0)


def _mm0_body(x_ref, w_ref, dinv_ref, y_ref):
    y_ref[...] = jnp.dot(x_ref[...], w_ref[...],
                         preferred_element_type=jnp.float32) * dinv_ref[...]


def _fused_body(acc_ref, dinv_ref, b_ref, g_ref, bt_ref, w_ref, y_ref):
    dinv = dinv_ref[...]
    t = acc_ref[...] * dinv + b_ref[...]
    t = jnp.maximum(t, 0.0) * (g_ref[...] * GS) + bt_ref[...]
    y_ref[...] = jnp.dot(t, w_ref[...],
                         preferred_element_type=jnp.float32) * dinv


def _final_body(acc_ref, dinv_ref, b_ref, g_ref, bt_ref, w_ref, lb_ref, o_ref):
    t = acc_ref[...] * dinv_ref[...] + b_ref[...]
    t = jnp.maximum(t, 0.0) * (g_ref[...] * GS) + bt_ref[...]
    o_ref[...] = jnp.dot(t, w_ref[...],
                         preferred_element_type=jnp.float32) + lb_ref[...]


BM = 400
GRID = (N // BM,)


def _row_spec(width):
    return pl.BlockSpec((BM, width), lambda i: (i, 0))


def _rep_spec(shape):
    return pl.BlockSpec(shape, lambda i: (0,) * len(shape))


def kernel(x, edge_index, conv_W, conv_b, bn_gamma, bn_beta, lin_W, lin_b):
    src = edge_index[0]
    dst = edge_index[1]
    E = src.shape[0]

    srcs_p, dstl_p, offs_p, cnts_p, deg_p = _make_partition(E)(src, dst)
    coffs_p, ncts_p = _make_chunks(E // NT + NT * CH)(offs_p, cnts_p)

    dinv_row = pl.pallas_call(
        _dinv_body,
        out_shape=jax.ShapeDtypeStruct((1, N), jnp.float32),
    )(deg_p)
    dinv = dinv_row.reshape(N, 1)

    mm0 = pl.pallas_call(
        _mm0_body,
        grid=GRID,
        in_specs=[_row_spec(D), _rep_spec((D, D)), _row_spec(1)],
        out_specs=_row_spec(D),
        out_shape=jax.ShapeDtypeStruct((N, D), jnp.float32),
    )
    fused = pl.pallas_call(
        _fused_body,
        grid=GRID,
        in_specs=[_row_spec(D), _row_spec(1), _rep_spec((1, D)),
                  _rep_spec((1, D)), _rep_spec((1, D)), _rep_spec((D, D))],
        out_specs=_row_spec(D),
        out_shape=jax.ShapeDtypeStruct((N, D), jnp.float32),
    )
    final = pl.pallas_call(
        _final_body,
        grid=GRID,
        in_specs=[_row_spec(D), _row_spec(1), _rep_spec((1, D)),
                  _rep_spec((1, D)), _rep_spec((1, D)), _rep_spec((D, OUT)),
                  _rep_spec((1, OUT))],
        out_specs=_row_spec(OUT),
        out_shape=jax.ShapeDtypeStruct((N, OUT), jnp.float32),
    )
    agg_call = _make_agg()

    y = mm0(x, conv_W[0], dinv)
    for i in range(L):
        acc = agg_call(y, srcs_p, dstl_p, coffs_p, ncts_p)
        if i < L - 1:
            y = fused(acc, dinv, conv_b[i].reshape(1, D),
                      bn_gamma[i].reshape(1, D), bn_beta[i].reshape(1, D),
                      conv_W[i + 1])
        else:
            out = final(acc, dinv, conv_b[i].reshape(1, D),
                        bn_gamma[i].reshape(1, D), bn_beta[i].reshape(1, D),
                        lin_W, lin_b.reshape(1, OUT))
    return out


# speedup vs baseline: 1.0543x; 1.0012x over previous
import jax
import jax.numpy as jnp
from jax import lax
from jax.experimental import pallas as pl
from jax.experimental.pallas import tpu as pltpu
from jax.experimental.pallas import tpu_sc as plsc

N = 10000
D = 256
OUT = 64
L = 10
EPS = 1e-5

NC = 2
NS = 16
NT = NC * NS
BUCKET = 320
ACC_ROWS = 336
DUMP = 328
CH = 16
GS = float(1.0 / (1.0 + EPS) ** 0.5)


def _mesh():
    return plsc.VectorSubcoreMesh(core_axis_name="c", subcore_axis_name="s",
                                  num_cores=NC, num_subcores=NS)


_SC_PARAMS = pltpu.CompilerParams(needs_layout_passes=False)


def _partition_body(src_hbm, dst_hbm, srcs_out, dstl_out, offs_out, cnts_out,
                    deg_out, src_v, dst_v, so, do_, hist, cnt_v, offs_v):
    E = src_hbm.shape[0]
    ept = E // NT
    slot = ept + NT * CH
    c = lax.axis_index("c")
    s = lax.axis_index("s")
    k = c * NS + s

    pltpu.sync_copy(src_hbm.at[pl.ds(k * ept, ept)], src_v.at[pl.ds(0, ept)])
    pltpu.sync_copy(dst_hbm.at[pl.ds(k * ept, ept)], dst_v.at[pl.ds(0, ept)])

    zeros_i = jnp.zeros((16,), jnp.int32)
    dump_i = jnp.full((16,), DUMP, jnp.int32)
    zeros_f = jnp.zeros((16,), jnp.float32)
    ones_f = jnp.ones((16,), jnp.float32)
    ones_i = jnp.ones((16,), jnp.int32)
    iota = lax.iota(jnp.int32, 16)

    def init_hist(i, _):
        hist[pl.ds(i * 16, 16)] = zeros_f
        return 0
    lax.fori_loop(0, N // 16, init_hist, 0)

    def init_slots(i, _):
        so[pl.ds(i * 16, 16)] = (i * 16 + iota) & 8191
        do_[pl.ds(i * 16, 16)] = dump_i
        return 0
    lax.fori_loop(0, slot // 16, init_slots, 0)

    cnt_v[pl.ds(0, 16)] = zeros_i
    cnt_v[pl.ds(16, 16)] = zeros_i

    def step(i, _):
        dstv = dst_v[pl.ds(i * 16, 16)]
        valid = iota < (ept - i * 16)
        dstv = jnp.where(valid, dstv, 0)
        plsc.addupdate_scatter(hist, [dstv], ones_f, mask=valid)
        plsc.addupdate_scatter(cnt_v, [dstv // BUCKET], ones_i, mask=valid)
        return 0
    lax.fori_loop(0, (ept + 15) // 16, step, 0)

    c0 = cnt_v[pl.ds(0, 16)]
    c1 = cnt_v[pl.ds(16, 16)]
    p0 = (c0 + (CH - 1)) & (-CH)
    p1 = (c1 + (CH - 1)) & (-CH)
    i0 = plsc.cumsum(p0)
    i1 = plsc.cumsum(p1) + jnp.max(i0)
    offs_v[pl.ds(0, 16)] = i0 - p0
    offs_v[pl.ds(16, 16)] = i1 - p1
    cnt_v[pl.ds(0, 16)] = p0
    cnt_v[pl.ds(16, 16)] = p1

    nsteps = (ept + 15) // 16

    def per_bucket(b, _):
        o0 = offs_v[pl.ds(b, 16)][0]

        def sweep(i, o):
            srcv = src_v[pl.ds(i * 16, 16)]
            dstv = dst_v[pl.ds(i * 16, 16)]
            valid = iota < (ept - i * 16)
            dstv = jnp.where(valid, dstv, 0)
            m = valid & ((dstv // BUCKET) == b)
            plsc.store_compressed(so.at[pl.ds(o, 16)], srcv, mask=m)
            plsc.store_compressed(do_.at[pl.ds(o, 16)], dstv - b * BUCKET,
                                  mask=m)
            return o + jnp.max(plsc.all_reduce_population_count(m))
        lax.fori_loop(0, nsteps, sweep, o0)
        return 0
    lax.fori_loop(0, 32, per_bucket, 0)

    pltpu.sync_copy(so, srcs_out.at[pl.ds(k * slot, slot)])
    pltpu.sync_copy(do_, dstl_out.at[pl.ds(k * slot, slot)])
    pltpu.sync_copy(offs_v.at[pl.ds(0, 32)], offs_out.at[pl.ds(k * 32, 32)])
    pltpu.sync_copy(cnt_v, cnts_out.at[pl.ds(k * 32, 32)])
    pltpu.sync_copy(hist, deg_out.at[k])


def _make_partition(E):
    ept = E // NT
    slot = ept + NT * CH
    return pl.kernel(
        _partition_body,
        out_type=[
            jax.ShapeDtypeStruct((NT * slot,), jnp.int32),
            jax.ShapeDtypeStruct((NT * slot,), jnp.int32),
            jax.ShapeDtypeStruct((NT * 32,), jnp.int32),
            jax.ShapeDtypeStruct((NT * 32,), jnp.int32),
            jax.ShapeDtypeStruct((NT, N), jnp.float32),
        ],
        mesh=_mesh(),
        compiler_params=_SC_PARAMS,
        scratch_types=[
            pltpu.VMEM((ept + 16,), jnp.int32),
            pltpu.VMEM((ept + 16,), jnp.int32),
            pltpu.VMEM((slot,), jnp.int32),
            pltpu.VMEM((slot,), jnp.int32),
            pltpu.VMEM((N,), jnp.float32),
            pltpu.VMEM((32,), jnp.int32),
            pltpu.VMEM((48,), jnp.int32),
        ],
    )


MAXCH = 10240


def _make_chunks(eslot):
    def body(offs_hbm, cnts_hbm, coffs_out, ncts_out, tabs, coffs_v):
        c = lax.axis_index("c")
        s = lax.axis_index("s")
        w = c * NS + s

        pltpu.sync_copy(offs_hbm, tabs.at[pl.ds(0, NT * 32)])
        pltpu.sync_copy(cnts_hbm, tabs.at[pl.ds(NT * 32, NT * 32)])

        iota = lax.iota(jnp.int32, 16)
        idx0 = iota * 32 + w
        colo0 = plsc.load_gather(tabs, [idx0])
        colo1 = plsc.load_gather(tabs, [idx0 + 512])
        coln0 = plsc.load_gather(tabs, [idx0 + NT * 32])
        coln1 = plsc.load_gather(tabs, [idx0 + NT * 32 + 512])

        def _extr(v, i):
            return jnp.max(jnp.where(iota == i, v, 0))

        def per_tile(t, ptr):
            o = jnp.where(t < NS, _extr(colo0, t), _extr(colo1, t - NS))
            n = jnp.where(t < NS, _extr(coln0, t), _extr(coln1, t - NS))
            base = t * eslot + o
            nch = n // CH

            def grp(g, pp):
                ii = g * 16 + iota
                plsc.store_scatter(coffs_v, [pp + ii], base + ii * CH,
                                   mask=ii < nch)
                return pp
            lax.fori_loop(0, (nch + 15) // 16, grp, ptr)
            return ptr + nch
        nct = lax.fori_loop(0, NT, per_tile, jnp.int32(0))

        pltpu.sync_copy(coffs_v, coffs_out.at[pl.ds(w * MAXCH, MAXCH)])
        tabs[pl.ds(0, 16)] = jnp.zeros((16,), jnp.int32) + nct
        pltpu.sync_copy(tabs.at[pl.ds(0, 16)], ncts_out.at[pl.ds(w * 16, 16)])

    return pl.kernel(
        body,
        out_type=[
            jax.ShapeDtypeStruct((NT * MAXCH,), jnp.int32),
            jax.ShapeDtypeStruct((NT * 16,), jnp.int32),
        ],
        mesh=_mesh(),
        compiler_params=_SC_PARAMS,
        scratch_types=[
            pltpu.VMEM((2 * NT * 32 + 16,), jnp.int32),
            pltpu.VMEM((MAXCH,), jnp.int32),
        ],
    )


def _agg_body(y_hbm, srcs_hbm, dstl_hbm, coffs_hbm, ncts_hbm, agg_hbm,
              acc, rows0, rows1, sidx0, sidx1, coffs_v, cnt_v,
              isem0, isem1, gsem0, gsem1, dsem0, dsem1):
    c = lax.axis_index("c")
    s = lax.axis_index("s")
    w = c * NS + s
    base_row = w * BUCKET

    pltpu.sync_copy(coffs_hbm.at[pl.ds(w * MAXCH, MAXCH)], coffs_v)
    pltpu.sync_copy(ncts_hbm.at[pl.ds(w * 16, 16)], cnt_v)
    nct = jnp.max(cnt_v[...])

    last = N - BUCKET * (NT - 1)
    extra = BUCKET - last
    pltpu.sync_copy(y_hbm.at[pl.ds(base_row, last)], acc.at[pl.ds(0, last)])
    @pl.when(w < NT - 1)
    def _():
        pltpu.sync_copy(y_hbm.at[pl.ds(base_row + last, extra)],
                        acc.at[pl.ds(last, extra)])

    rows_ = (rows0, rows1)
    ibuf_ = (sidx0, sidx1)
    isem_ = (isem0, isem1)
    gsem_ = (gsem0, gsem1)
    dsem_ = (dsem0, dsem1)

    def _start_idx(j, b):
        v = plsc.load_gather(coffs_v, [jnp.full((16,), j, jnp.int32)])
        off = pl.multiple_of(v[0], 8)
        pltpu.async_copy(srcs_hbm.at[pl.ds(off, CH)],
                         ibuf_[b].at[pl.ds(0, CH)], isem_[b])
        pltpu.async_copy(dstl_hbm.at[pl.ds(off, CH)],
                         ibuf_[b].at[pl.ds(CH, CH)], dsem_[b])

    def _wait_idx(b):
        pltpu.make_async_copy(srcs_hbm.at[pl.ds(0, CH)],
                              ibuf_[b].at[pl.ds(0, CH)], isem_[b]).wait()

    def _wait_didx(b):
        pltpu.make_async_copy(dstl_hbm.at[pl.ds(0, CH)],
                              ibuf_[b].at[pl.ds(CH, CH)], dsem_[b]).wait()

    @pl.when(nct > 0)
    def _():
        _start_idx(0, 0)

    def pair(i, _):
        for b in (0, 1):
            j = 2 * i + b
            nb = 1 - b

            @pl.when(j < nct)
            def _():
                _wait_idx(b)
                pltpu.async_copy(y_hbm.at[ibuf_[b].at[pl.ds(0, CH)]],
                                 rows_[b], gsem_[b])

            if b == 0:
                @pl.when((j == 0) & (nct > 1))
                def _():
                    _start_idx(1, 1)

            @pl.when((j > 0) & (j <= nct))
            def _():
                pltpu.make_async_copy(y_hbm.at[pl.ds(0, CH)], rows_[nb],
                                      gsem_[nb]).wait()
                _wait_didx(nb)
                dvecs = [ibuf_[nb][pl.ds(CH + k * 16, 16)]
                         for k in range(CH // 16)]

                @pl.when(j + 1 < nct)
                def _():
                    _start_idx(j + 1, nb)

                rnb = rows_[nb]
                for k in range(CH // 16):
                    dvec = dvecs[k]
                    for e in range(16):
                        dl = dvec[e]
                        r = k * 16 + e
                        for jj in range(D // 16):
                            plsc.addupdate(acc.at[dl, pl.ds(jj * 16, 16)],
                                           rnb[r, pl.ds(jj * 16, 16)])
        return 0
    lax.fori_loop(0, (nct + 2) // 2, pair, 0)

    pltpu.sync_copy(acc.at[pl.ds(0, last)],
                    agg_hbm.at[pl.ds(base_row, last)])
    @pl.when(w < NT - 1)
    def _():
        pltpu.sync_copy(acc.at[pl.ds(last, extra)],
                        agg_hbm.at[pl.ds(base_row + last, extra)])


def _make_agg():
    return pl.kernel(
        _agg_body,
        out_type=jax.ShapeDtypeStruct((N, D), jnp.float32),
        mesh=_mesh(),
        compiler_params=_SC_PARAMS,
        scratch_types=[
            pltpu.VMEM((ACC_ROWS, D), jnp.float32),
            pltpu.VMEM((CH, D), jnp.float32),
            pltpu.VMEM((CH, D), jnp.float32),
            pltpu.VMEM((2 * CH,), jnp.int32),
            pltpu.VMEM((2 * CH,), jnp.int32),
            pltpu.VMEM((MAXCH,), jnp.int32),
            pltpu.VMEM((16,), jnp.int32),
            pltpu.SemaphoreType.DMA,
            pltpu.SemaphoreType.DMA,
            pltpu.SemaphoreType.DMA,
            pltpu.SemaphoreType.DMA,
            pltpu.SemaphoreType.DMA,
            pltpu.SemaphoreType.DMA,
        ],
    )


def _dinv_body(dp_ref, o_ref):
    o_ref[...] = lax.rsqrt(jnp.sum(dp_ref[...], axis=0, keepdims=True) + 1.0)


def _mm0_body(x_ref, w_ref, dinv_ref, y_ref):
    y_ref[...] = jnp.dot(x_ref[...], w_ref[...],
                         preferred_element_type=jnp.float32) * dinv_ref[...]


def _fused_body(acc_ref, dinv_ref, b_ref, g_ref, bt_ref, w_ref, y_ref):
    dinv = dinv_ref[...]
    t = acc_ref[...] * dinv + b_ref[...]
    t = jnp.maximum(t, 0.0) * (g_ref[...] * GS) + bt_ref[...]
    y_ref[...] = jnp.dot(t, w_ref[...],
                         preferred_element_type=jnp.float32) * dinv


def _final_body(acc_ref, dinv_ref, b_ref, g_ref, bt_ref, w_ref, lb_ref, o_ref):
    t = acc_ref[...] * dinv_ref[...] + b_ref[...]
    t = jnp.maximum(t, 0.0) * (g_ref[...] * GS) + bt_ref[...]
    o_ref[...] = jnp.dot(t, w_ref[...],
                         preferred_element_type=jnp.float32) + lb_ref[...]


BM = 400
GRID = (N // BM,)


def _row_spec(width):
    return pl.BlockSpec((BM, width), lambda i: (i, 0))


def _rep_spec(shape):
    return pl.BlockSpec(shape, lambda i: (0,) * len(shape))


def kernel(x, edge_index, conv_W, conv_b, bn_gamma, bn_beta, lin_W, lin_b):
    src = edge_index[0]
    dst = edge_index[1]
    E = src.shape[0]

    srcs_p, dstl_p, offs_p, cnts_p, deg_p = _make_partition(E)(src, dst)
    coffs_p, ncts_p = _make_chunks(E // NT + NT * CH)(offs_p, cnts_p)

    dinv_row = pl.pallas_call(
        _dinv_body,
        out_shape=jax.ShapeDtypeStruct((1, N), jnp.float32),
    )(deg_p)
    dinv = dinv_row.reshape(N, 1)

    mm0 = pl.pallas_call(
        _mm0_body,
        grid=GRID,
        in_specs=[_row_spec(D), _rep_spec((D, D)), _row_spec(1)],
        out_specs=_row_spec(D),
        out_shape=jax.ShapeDtypeStruct((N, D), jnp.float32),
    )
    fused = pl.pallas_call(
        _fused_body,
        grid=GRID,
        in_specs=[_row_spec(D), _row_spec(1), _rep_spec((1, D)),
                  _rep_spec((1, D)), _rep_spec((1, D)), _rep_spec((D, D))],
        out_specs=_row_spec(D),
        out_shape=jax.ShapeDtypeStruct((N, D), jnp.float32),
    )
    final = pl.pallas_call(
        _final_body,
        grid=GRID,
        in_specs=[_row_spec(D), _row_spec(1), _rep_spec((1, D)),
                  _rep_spec((1, D)), _rep_spec((1, D)), _rep_spec((D, OUT)),
                  _rep_spec((1, OUT))],
        out_specs=_row_spec(OUT),
        out_shape=jax.ShapeDtypeStruct((N, OUT), jnp.float32),
    )
    agg_call = _make_agg()

    y = mm0(x, conv_W[0], dinv)
    for i in range(L):
        acc = agg_call(y, srcs_p, dstl_p, coffs_p, ncts_p)
        if i < L - 1:
            y = fused(acc, dinv, conv_b[i].reshape(1, D),
                      bn_gamma[i].reshape(1, D), bn_beta[i].reshape(1, D),
                      conv_W[i + 1])
        else:
            out = final(acc, dinv, conv_b[i].reshape(1, D),
                        bn_gamma[i].reshape(1, D), bn_beta[i].reshape(1, D),
                        lin_W, lin_b.reshape(1, OUT))
    return out
